# baseline (speedup 1.0000x reference)
import jax
import jax.numpy as jnp
from jax.experimental import pallas as pl
from jax.experimental.pallas import tpu as pltpu

B, F, NA, C, NO = 8, 6, 16384, 21, 16
THRESHOLD = 0.5
V0, V1 = 0.1, 0.2
NEG_POS_RATIO = 3
PR, PC = 128, 128
CH = 32
NCH = PR // CH


def _body(t_ref, a_ref, loc_ref, conf_ref, ll_ref, lc_ref,
          mine_s, npos_s, btov_s, bti_s):
    f32 = jnp.float32
    ts = [[t_ref[0, 0, o, c] for c in range(5)] for o in range(NO)]
    areat = [(ts[o][2] - ts[o][0]) * (ts[o][3] - ts[o][1]) for o in range(NO)]

    def anch(s):
        sl = pl.ds(s * CH, CH)
        return a_ref[0, sl, :], a_ref[1, sl, :], a_ref[2, sl, :], a_ref[3, sl, :]

    def chunk_iota(s):
        r0 = jax.lax.broadcasted_iota(jnp.int32, (CH, PC), 0)
        c0 = jax.lax.broadcasted_iota(jnp.int32, (CH, PC), 1)
        return (r0 + s * CH) * PC + c0

    mval = [None] * NO
    midx = [None] * NO
    for s in range(NCH):
        sl = pl.ds(s * CH, CH)
        acx, acy, aw, ah = anch(s)
        ax1 = acx - aw / 2.0
        ay1 = acy - ah / 2.0
        ax2 = acx + aw / 2.0
        ay2 = acy + ah / 2.0
        area_a = (ax2 - ax1) * (ay2 - ay1)
        aidx = chunk_iota(s)
        btov = jnp.full((CH, PC), -1.0, f32)
        bti = jnp.zeros((CH, PC), jnp.int32)
        for o in range(NO):
            tx1, ty1, tx2, ty2, _ = ts[o]
            iw = jnp.clip(jnp.minimum(ax2, tx2) - jnp.maximum(ax1, tx1), 0.0, None)
            ih = jnp.clip(jnp.minimum(ay2, ty2) - jnp.maximum(ay1, ty1), 0.0, None)
            inter = iw * ih
            ov = inter / (areat[o] + area_a - inter)
            upd = ov > btov
            btov = jnp.where(upd, ov, btov)
            bti = jnp.where(upd, o, bti)
            cm = jnp.max(ov)
            ci = jnp.min(jnp.where(ov == cm, aidx, NA))
            if s == 0:
                mval[o], midx[o] = cm, ci
            else:
                better = cm > mval[o]
                midx[o] = jnp.where(better, ci, midx[o])
                mval[o] = jnp.maximum(cm, mval[o])
        btov_s[sl] = btov
        bti_s[sl] = bti

    r = pl.program_id(0) * F + pl.program_id(1)
    hub = jnp.float32(0.0)
    cepos = jnp.float32(0.0)
    npos = jnp.float32(0.0)
    for s in range(NCH):
        sl = pl.ds(s * CH, CH)
        acx, acy, aw, ah = anch(s)
        aidx = chunk_iota(s)
        btov = btov_s[sl]
        bti = bti_s[sl]
        fm = aidx == midx[0]
        bti = jnp.where(fm, 0, bti)
        for o in range(1, NO):
            hit = aidx == midx[o]
            fm = jnp.logical_or(fm, hit)
            bti = jnp.where(hit, o, bti)
        btov = jnp.where(fm, 2.0, btov)
        mx1 = jnp.zeros((CH, PC), f32)
        my1 = jnp.zeros((CH, PC), f32)
        mx2 = jnp.zeros((CH, PC), f32)
        my2 = jnp.zeros((CH, PC), f32)
        lab = jnp.zeros((CH, PC), f32)
        for o in range(NO):
            sel = bti == o
            mx1 = jnp.where(sel, ts[o][0], mx1)
            my1 = jnp.where(sel, ts[o][1], my1)
            mx2 = jnp.where(sel, ts[o][2], mx2)
            my2 = jnp.where(sel, ts[o][3], my2)
            lab = jnp.where(sel, ts[o][4], lab)
        pos = btov >= THRESHOLD
        posf = pos.astype(f32)
        cls = jnp.where(pos, lab.astype(jnp.int32) + 1, 0)
        gcx = ((mx1 + mx2) / 2.0 - acx) / (V0 * aw)
        gcy = ((my1 + my2) / 2.0 - acy) / (V0 * ah)
        gw = jnp.log(jnp.clip(mx2 - mx1, 1e-6, None) / aw) / V1
        gh = jnp.log(jnp.clip(my2 - my1, 1e-6, None) / ah) / V1
        for c, g in enumerate((gcx, gcy, gw, gh)):
            d = loc_ref[0, 0, c, sl, :] - g
            ad = jnp.abs(d)
            h = jnp.where(ad < 1.0, 0.5 * d * d, ad - 0.5)
            hub = hub + jnp.sum(h * posf)
        mx = conf_ref[0, 0, 0, sl, :]
        for c in range(1, C):
            mx = jnp.maximum(mx, conf_ref[0, 0, c, sl, :])
        se = jnp.zeros((CH, PC), f32)
        tl = jnp.zeros((CH, PC), f32)
        for c in range(C):
            x = conf_ref[0, 0, c, sl, :]
            se = se + jnp.exp(x - mx)
            tl = jnp.where(cls == c, x, tl)
        ce = mx + jnp.log(se) - tl
        cepos = cepos + jnp.sum(ce * posf)
        npos = npos + jnp.sum(posf)
        mine_s[pl.ds(r, 1), sl] = jnp.where(pos, 0.0, ce).reshape(1, CH, PC)

    npos_s[pl.ds(r, 1)] = jnp.full((1, PC), npos, f32)

    @pl.when((pl.program_id(0) == 0) & (pl.program_id(1) == 0))
    def _init():
        ll_ref[...] = jnp.zeros_like(ll_ref)
        lc_ref[...] = jnp.zeros_like(lc_ref)

    ll_ref[...] += hub
    lc_ref[...] += cepos

    @pl.when((pl.program_id(0) == B - 1) & (pl.program_id(1) == F - 1))
    def _mine_all():
        nrow = B * F
        kv = jnp.minimum(npos_s[:, 0:1].reshape(nrow, 1, 1) * NEG_POS_RATIO,
                         float(NA - 1))
        mall = mine_s[...]
        mb = jax.lax.bitcast_convert_type(mall, jnp.int32)

        def rsum(x):
            return jnp.sum(jnp.sum(x, axis=1, keepdims=True), axis=2,
                           keepdims=True)

        def bis(_, lohi):
            lo, hi = lohi
            mid = lo + (hi - lo) // 2
            cnt = rsum(jnp.where(mb >= mid, 1.0, 0.0))
            ok = cnt >= kv
            return (jnp.where(ok, mid, lo), jnp.where(ok, hi, mid))

        lo0 = jnp.zeros((nrow, 1, 1), jnp.int32)
        hi0 = jnp.full((nrow, 1, 1), 0x7F800000, jnp.int32)
        lo, _ = jax.lax.fori_loop(0, 31, bis, (lo0, hi0))
        vkth = jax.lax.bitcast_convert_type(lo, f32)
        gtm = mall > vkth
        cgt = rsum(jnp.where(gtm, 1.0, 0.0))
        sgt = rsum(jnp.where(gtm, mall, 0.0))
        topk = sgt + (kv - cgt) * vkth
        topk = jnp.where(kv > 0, topk, 0.0)
        lc_ref[...] += jnp.sum(topk)


def kernel(loc_data, conf_data, anchors, targets):
    loc_p = loc_data.reshape(B, F, NA, 4).transpose(0, 1, 3, 2).reshape(B, F, 4, PR, PC)
    conf_p = conf_data.reshape(B, F, NA, C).transpose(0, 1, 3, 2).reshape(B, F, C, PR, PC)
    anch_p = anchors.T.reshape(4, PR, PC)
    ll, lc = pl.pallas_call(
        _body,
        grid=(B, F),
        in_specs=[
            pl.BlockSpec((1, 1, NO, 5), lambda b, f: (b, f, 0, 0)),
            pl.BlockSpec((4, PR, PC), lambda b, f: (0, 0, 0)),
            pl.BlockSpec((1, 1, 4, PR, PC), lambda b, f: (b, f, 0, 0, 0)),
            pl.BlockSpec((1, 1, C, PR, PC), lambda b, f: (b, f, 0, 0, 0)),
        ],
        out_specs=[
            pl.BlockSpec((1, 1), lambda b, f: (0, 0)),
            pl.BlockSpec((1, 1), lambda b, f: (0, 0)),
        ],
        out_shape=[
            jax.ShapeDtypeStruct((1, 1), jnp.float32),
            jax.ShapeDtypeStruct((1, 1), jnp.float32),
        ],
        scratch_shapes=[
            pltpu.VMEM((B * F, PR, PC), jnp.float32),
            pltpu.VMEM((B * F, PC), jnp.float32),
            pltpu.VMEM((PR, PC), jnp.float32),
            pltpu.VMEM((PR, PC), jnp.int32),
        ],
    )(targets, anch_p, loc_p, conf_p)
    return (ll[0, 0], lc[0, 0])


# speedup vs baseline: 3.0304x; 3.0304x over previous
import jax
import jax.numpy as jnp
from jax.experimental import pallas as pl
from jax.experimental.pallas import tpu as pltpu

B, F, NA, C, NO = 8, 6, 16384, 21, 16
THRESHOLD = 0.5
V0, V1 = 0.1, 0.2
NEG_POS_RATIO = 3
PR, PC = 128, 128
CH = 32
NCH = PR // CH


def _body(t_ref, a_ref, loc_ref, conf_ref, ll_ref, lc_ref,
          mine_s, npos_s, btov_s, bti_s):
    f32 = jnp.float32
    ts = [[t_ref[0, 0, o, c] for c in range(5)] for o in range(NO)]
    areat = [(ts[o][2] - ts[o][0]) * (ts[o][3] - ts[o][1]) for o in range(NO)]

    def anch(s):
        sl = pl.ds(s * CH, CH)
        return a_ref[0, sl, :], a_ref[1, sl, :], a_ref[2, sl, :], a_ref[3, sl, :]

    def chunk_iota(s):
        r0 = jax.lax.broadcasted_iota(jnp.int32, (CH, PC), 0)
        c0 = jax.lax.broadcasted_iota(jnp.int32, (CH, PC), 1)
        return (r0 + s * CH) * PC + c0

    pmax = [None] * NO
    psub = [None] * NO
    rio = jax.lax.broadcasted_iota(jnp.int32, (CH, PC), 0)
    for s in range(NCH):
        sl = pl.ds(s * CH, CH)
        acx, acy, aw, ah = anch(s)
        ax1 = acx - aw / 2.0
        ay1 = acy - ah / 2.0
        ax2 = acx + aw / 2.0
        ay2 = acy + ah / 2.0
        area_a = (ax2 - ax1) * (ay2 - ay1)
        btov = jnp.full((CH, PC), -1.0, f32)
        bti = jnp.zeros((CH, PC), jnp.int32)
        for o in range(NO):
            tx1, ty1, tx2, ty2, _ = ts[o]
            iw = jnp.clip(jnp.minimum(ax2, tx2) - jnp.maximum(ax1, tx1), 0.0, None)
            ih = jnp.clip(jnp.minimum(ay2, ty2) - jnp.maximum(ay1, ty1), 0.0, None)
            inter = iw * ih
            ov = inter / (areat[o] + area_a - inter)
            upd = ov > btov
            btov = jnp.where(upd, ov, btov)
            bti = jnp.where(upd, o, bti)
            pm = jnp.max(ov, axis=0, keepdims=True)
            sm = jnp.min(jnp.where(ov == pm, rio, CH), axis=0,
                         keepdims=True) + s * CH
            if s == 0:
                pmax[o], psub[o] = pm, sm
            else:
                better = pm > pmax[o]
                psub[o] = jnp.where(better, sm, psub[o])
                pmax[o] = jnp.where(better, pm, pmax[o])
        btov_s[sl] = btov
        bti_s[sl] = bti

    pcat = jnp.concatenate(pmax, axis=0)
    scat = jnp.concatenate(psub, axis=0)
    m16 = jnp.max(pcat, axis=1, keepdims=True)
    lane16 = jax.lax.broadcasted_iota(jnp.int32, (NO, PC), 1)
    bpi_vec = jnp.min(jnp.where(pcat == m16, scat * PC + lane16, NA),
                      axis=1, keepdims=True)
    midx = [bpi_vec[o, 0] for o in range(NO)]

    r = pl.program_id(0) * F + pl.program_id(1)
    hub = jnp.float32(0.0)
    cepos = jnp.float32(0.0)
    npos = jnp.float32(0.0)
    for s in range(NCH):
        sl = pl.ds(s * CH, CH)
        acx, acy, aw, ah = anch(s)
        aidx = chunk_iota(s)
        btov = btov_s[sl]
        bti = bti_s[sl]
        fm = aidx == midx[0]
        bti = jnp.where(fm, 0, bti)
        for o in range(1, NO):
            hit = aidx == midx[o]
            fm = jnp.logical_or(fm, hit)
            bti = jnp.where(hit, o, bti)
        btov = jnp.where(fm, 2.0, btov)
        mx1 = jnp.zeros((CH, PC), f32)
        my1 = jnp.zeros((CH, PC), f32)
        mx2 = jnp.zeros((CH, PC), f32)
        my2 = jnp.zeros((CH, PC), f32)
        lab = jnp.zeros((CH, PC), f32)
        for o in range(NO):
            sel = bti == o
            mx1 = jnp.where(sel, ts[o][0], mx1)
            my1 = jnp.where(sel, ts[o][1], my1)
            mx2 = jnp.where(sel, ts[o][2], mx2)
            my2 = jnp.where(sel, ts[o][3], my2)
            lab = jnp.where(sel, ts[o][4], lab)
        pos = btov >= THRESHOLD
        posf = pos.astype(f32)
        cls = jnp.where(pos, lab.astype(jnp.int32) + 1, 0)
        gcx = ((mx1 + mx2) / 2.0 - acx) / (V0 * aw)
        gcy = ((my1 + my2) / 2.0 - acy) / (V0 * ah)
        gw = jnp.log(jnp.clip(mx2 - mx1, 1e-6, None) / aw) / V1
        gh = jnp.log(jnp.clip(my2 - my1, 1e-6, None) / ah) / V1
        for c, g in enumerate((gcx, gcy, gw, gh)):
            d = loc_ref[0, 0, c, sl, :] - g
            ad = jnp.abs(d)
            h = jnp.where(ad < 1.0, 0.5 * d * d, ad - 0.5)
            hub = hub + jnp.sum(h * posf)
        mx = conf_ref[0, 0, 0, sl, :]
        for c in range(1, C):
            mx = jnp.maximum(mx, conf_ref[0, 0, c, sl, :])
        se = jnp.zeros((CH, PC), f32)
        tl = jnp.zeros((CH, PC), f32)
        for c in range(C):
            x = conf_ref[0, 0, c, sl, :]
            se = se + jnp.exp(x - mx)
            tl = jnp.where(cls == c, x, tl)
        ce = mx + jnp.log(se) - tl
        cepos = cepos + jnp.sum(ce * posf)
        npos = npos + jnp.sum(posf)
        mine_s[pl.ds(r, 1), sl] = jnp.where(pos, 0.0, ce).reshape(1, CH, PC)

    npos_s[pl.ds(r, 1)] = jnp.full((1, PC), npos, f32)

    @pl.when((pl.program_id(0) == 0) & (pl.program_id(1) == 0))
    def _init():
        ll_ref[...] = jnp.zeros_like(ll_ref)
        lc_ref[...] = jnp.zeros_like(lc_ref)

    ll_ref[...] += hub
    lc_ref[...] += cepos

    @pl.when((pl.program_id(0) == B - 1) & (pl.program_id(1) == F - 1))
    def _mine_all():
        nrow = B * F
        kv = jnp.minimum(npos_s[:, 0:1].reshape(nrow, 1, 1) * NEG_POS_RATIO,
                         float(NA - 1))
        mall = mine_s[...]
        mb = jax.lax.bitcast_convert_type(mall, jnp.int32)

        def rsum(x):
            return jnp.sum(jnp.sum(x, axis=1, keepdims=True), axis=2,
                           keepdims=True)

        def bis(_, lohi):
            lo, hi = lohi
            mid = lo + (hi - lo) // 2
            cnt = rsum(jnp.where(mb >= mid, 1.0, 0.0))
            ok = cnt >= kv
            return (jnp.where(ok, mid, lo), jnp.where(ok, hi, mid))

        lo0 = jnp.zeros((nrow, 1, 1), jnp.int32)
        hi0 = jnp.full((nrow, 1, 1), 0x7F800000, jnp.int32)
        lo, _ = jax.lax.fori_loop(0, 31, bis, (lo0, hi0))
        vkth = jax.lax.bitcast_convert_type(lo, f32)
        gtm = mall > vkth
        cgt = rsum(jnp.where(gtm, 1.0, 0.0))
        sgt = rsum(jnp.where(gtm, mall, 0.0))
        topk = sgt + (kv - cgt) * vkth
        topk = jnp.where(kv > 0, topk, 0.0)
        lc_ref[...] += jnp.sum(topk)


def kernel(loc_data, conf_data, anchors, targets):
    loc_p = loc_data.reshape(B, F, NA, 4).transpose(0, 1, 3, 2).reshape(B, F, 4, PR, PC)
    conf_p = conf_data.reshape(B, F, NA, C).transpose(0, 1, 3, 2).reshape(B, F, C, PR, PC)
    anch_p = anchors.T.reshape(4, PR, PC)
    ll, lc = pl.pallas_call(
        _body,
        grid=(B, F),
        in_specs=[
            pl.BlockSpec((1, 1, NO, 5), lambda b, f: (b, f, 0, 0)),
            pl.BlockSpec((4, PR, PC), lambda b, f: (0, 0, 0)),
            pl.BlockSpec((1, 1, 4, PR, PC), lambda b, f: (b, f, 0, 0, 0)),
            pl.BlockSpec((1, 1, C, PR, PC), lambda b, f: (b, f, 0, 0, 0)),
        ],
        out_specs=[
            pl.BlockSpec((1, 1), lambda b, f: (0, 0)),
            pl.BlockSpec((1, 1), lambda b, f: (0, 0)),
        ],
        out_shape=[
            jax.ShapeDtypeStruct((1, 1), jnp.float32),
            jax.ShapeDtypeStruct((1, 1), jnp.float32),
        ],
        scratch_shapes=[
            pltpu.VMEM((B * F, PR, PC), jnp.float32),
            pltpu.VMEM((B * F, PC), jnp.float32),
            pltpu.VMEM((PR, PC), jnp.float32),
            pltpu.VMEM((PR, PC), jnp.int32),
        ],
    )(targets, anch_p, loc_p, conf_p)
    return (ll[0, 0], lc[0, 0])
